# baseline (speedup 1.0000x reference)
import functools

import jax
import jax.numpy as jnp
from jax import lax
from jax.experimental import pallas as pl
from jax.experimental.pallas import tpu as pltpu
from jax.experimental.pallas import tpu_sc as plsc

N = 100000
E = 200000
NC = 2
NS = 16
NW = NC * NS

LINK_CHUNK = 6272
NSUB = 4
SUBL = LINK_CHUNK // NSUB
HALF_N = N // 2
NODE_CHUNK = 3136


@functools.cache
def _mesh():
    return plsc.VectorSubcoreMesh(core_axis_name="c", subcore_axis_name="s",
                                  num_cores=NC, num_subcores=NS)


def _body(disch, status, head, tail, over, links, melt,
          out, signed, flags,
          d_v, s_v, h_v, t_v, oh_v, ot_v, o_v,
          idx_v, g_v, m_v, r_v, f_v,
          sem_in, sem_b, g0, g1, g2, g3, sem_out):
    sc = lax.axis_index("c")
    sid = lax.axis_index("s")
    wid = sid * NC + sc

    f_v[...] = jnp.zeros((16,), jnp.int32)
    pltpu.sync_copy(f_v, flags.at[pl.ds(sc * 16, 16)])

    base = jnp.minimum(wid * LINK_CHUNK, E - LINK_CHUNK)
    base = pl.multiple_of(base, 8)
    sl = pl.ds(base, LINK_CHUNK)
    cp_h = pltpu.async_copy(head.at[sl], h_v, sem_in)
    cp_t = pltpu.async_copy(tail.at[sl], t_v, sem_in)
    cp_d = pltpu.async_copy(disch.at[sl], d_v, sem_in)
    cp_s = pltpu.async_copy(status.at[sl], s_v, sem_in)

    nbase = sc * HALF_N + jnp.minimum(sid * NODE_CHUNK, HALF_N - NODE_CHUNK)
    nbase = pl.multiple_of(nbase, 8)
    lb = pl.multiple_of(nbase * 4, 8)
    cp_i = pltpu.async_copy(links.at[pl.ds(lb, NODE_CHUNK * 4)], idx_v, sem_b)
    cp_m = pltpu.async_copy(melt.at[pl.ds(nbase, NODE_CHUNK)], m_v, sem_b)

    cp_h.wait()
    cp_t.wait()

    gsem = [g0, g1, g2, g3]
    pend = []
    for c in range(NSUB):
        cs = pl.ds(c * SUBL, SUBL)
        pend.append((
            pltpu.async_copy(over.at[h_v.at[cs]], oh_v.at[cs], gsem[c]),
            pltpu.async_copy(over.at[t_v.at[cs]], ot_v.at[cs], gsem[c]),
        ))
    cp_d.wait()
    cp_s.wait()

    wb = []
    for c in range(NSUB):
        cg1, cg2 = pend[c]
        cg1.wait()
        cg2.wait()

        def body(i, carry, c=c):
            v = pl.ds(c * SUBL + i * 16, 16)
            d = jnp.where(s_v[v] == 4, 0.0, d_v[v])
            o_v[v] = jnp.where(oh_v[v] > ot_v[v], d, -d)
            return carry

        lax.fori_loop(0, SUBL // 16, body, 0)
        wb.append(pltpu.async_copy(
            o_v.at[pl.ds(c * SUBL, SUBL)],
            signed.at[pl.ds(base + c * SUBL, SUBL)], sem_out))
    for cp in wb:
        cp.wait()

    plsc.subcore_barrier()

    f_v[...] = jnp.ones((16,), jnp.int32)
    pltpu.sync_copy(f_v, flags.at[pl.ds(sc * 16, 16)])

    other = pl.multiple_of((1 - sc) * 16, 16)
    f_v[...] = jnp.zeros((16,), jnp.int32)

    def _poll(i, carry):
        @pl.when(f_v[pl.ds(0, 16)][0] == 0)
        def _again():
            pltpu.async_copy(flags.at[pl.ds(other, 16)], f_v, sem_in).wait()
        return carry

    lax.fori_loop(0, 128, _poll, 0)

    cp_i.wait()
    cp_g = pltpu.async_copy(signed.at[idx_v], g_v, g0)
    cp_m.wait()
    cp_g.wait()

    lane = lax.iota(jnp.int32, 16)

    def bodyb(j, carry):
        nb = j * 16
        i0 = (nb + lane) * 4
        acc = plsc.load_gather(g_v, [i0])
        acc = acc + plsc.load_gather(g_v, [i0 + 1])
        acc = acc + plsc.load_gather(g_v, [i0 + 2])
        acc = acc + plsc.load_gather(g_v, [i0 + 3])
        v = pl.ds(nb, 16)
        r_v[v] = acc - m_v[v]
        return carry

    lax.fori_loop(0, NODE_CHUNK // 16, bodyb, 0)

    pltpu.sync_copy(r_v, out.at[pl.ds(nbase, NODE_CHUNK)])


@functools.cache
def _fused_call():
    return pl.kernel(
        _body,
        out_type=(jax.ShapeDtypeStruct((N,), jnp.float32),
                  jax.ShapeDtypeStruct((E,), jnp.float32),
                  jax.ShapeDtypeStruct((NC * 16,), jnp.int32)),
        mesh=_mesh(),
        compiler_params=pltpu.CompilerParams(needs_layout_passes=False),
        scratch_types=[
            pltpu.VMEM((LINK_CHUNK,), jnp.float32),
            pltpu.VMEM((LINK_CHUNK,), jnp.int32),
            pltpu.VMEM((LINK_CHUNK,), jnp.int32),
            pltpu.VMEM((LINK_CHUNK,), jnp.int32),
            pltpu.VMEM((LINK_CHUNK,), jnp.float32),
            pltpu.VMEM((LINK_CHUNK,), jnp.float32),
            pltpu.VMEM((LINK_CHUNK,), jnp.float32),
            pltpu.VMEM((NODE_CHUNK * 4,), jnp.int32),
            pltpu.VMEM((NODE_CHUNK * 4,), jnp.float32),
            pltpu.VMEM((NODE_CHUNK,), jnp.float32),
            pltpu.VMEM((NODE_CHUNK,), jnp.float32),
            pltpu.VMEM((16,), jnp.int32),
            pltpu.SemaphoreType.DMA,
            pltpu.SemaphoreType.DMA,
            pltpu.SemaphoreType.DMA,
            pltpu.SemaphoreType.DMA,
            pltpu.SemaphoreType.DMA,
            pltpu.SemaphoreType.DMA,
            pltpu.SemaphoreType.DMA,
        ],
    )


def kernel(discharge, overburden, melt_rate, status_at_link,
           node_at_link_head, node_at_link_tail, links_at_node):
    status = status_at_link.astype(jnp.int32)
    head = node_at_link_head.astype(jnp.int32)
    tail = node_at_link_tail.astype(jnp.int32)
    links = links_at_node.astype(jnp.int32).reshape(4 * N)
    out, _signed, _flags = _fused_call()(
        discharge, status, head, tail, overburden, links, melt_rate)
    return out


# speedup vs baseline: 1.9378x; 1.9378x over previous
import functools

import jax
import jax.numpy as jnp
from jax import lax
from jax.experimental import pallas as pl
from jax.experimental.pallas import tpu as pltpu
from jax.experimental.pallas import tpu_sc as plsc

N = 100000
E = 200000
NC = 2
NS = 16
NW = NC * NS

LINK_CHUNK = 6272
NSUB = 4
SUBL = LINK_CHUNK // NSUB
HALF_N = N // 2
NODE_CHUNK = 3136


@functools.cache
def _mesh():
    return plsc.VectorSubcoreMesh(core_axis_name="c", subcore_axis_name="s",
                                  num_cores=NC, num_subcores=NS)


def _body(disch, status, head, tail, over, links_t, melt,
          out, signed, flags,
          d_v, s_v, h_v, t_v, oh_v, ot_v, o_v,
          idx_v, g_v, m_v, r_v, f_v,
          sem_in, sem_b, g0, g1, g2, g3, sem_out):
    sc = lax.axis_index("c")
    sid = lax.axis_index("s")
    wid = sid * NC + sc

    f_v[...] = jnp.zeros((16,), jnp.int32)
    pltpu.sync_copy(f_v, flags.at[pl.ds(sc * 16, 16)])

    base = jnp.minimum(wid * LINK_CHUNK, E - LINK_CHUNK)
    base = pl.multiple_of(base, 8)
    sl = pl.ds(base, LINK_CHUNK)
    cp_h = pltpu.async_copy(head.at[sl], h_v, sem_in)
    cp_t = pltpu.async_copy(tail.at[sl], t_v, sem_in)
    cp_d = pltpu.async_copy(disch.at[sl], d_v, sem_in)
    cp_s = pltpu.async_copy(status.at[sl], s_v, sem_in)

    nbase = sc * HALF_N + jnp.minimum(sid * NODE_CHUNK, HALF_N - NODE_CHUNK)
    nbase = pl.multiple_of(nbase, 8)
    icp = []
    for l in range(4):
        icp.append(pltpu.async_copy(
            links_t.at[pl.ds(l * N + nbase, NODE_CHUNK)],
            idx_v.at[pl.ds(l * NODE_CHUNK, NODE_CHUNK)], sem_b))
    cp_m = pltpu.async_copy(melt.at[pl.ds(nbase, NODE_CHUNK)], m_v, sem_b)

    cp_h.wait()
    cp_t.wait()

    gsem = [g0, g1, g2, g3]
    pend = []
    for c in range(NSUB):
        cs = pl.ds(c * SUBL, SUBL)
        pend.append((
            pltpu.async_copy(over.at[h_v.at[cs]], oh_v.at[cs], gsem[c]),
            pltpu.async_copy(over.at[t_v.at[cs]], ot_v.at[cs], gsem[c]),
        ))
    cp_d.wait()
    cp_s.wait()

    wb = []
    for c in range(NSUB):
        cg1, cg2 = pend[c]
        cg1.wait()
        cg2.wait()

        def body(i, carry, c=c):
            v = pl.ds(c * SUBL + i * 16, 16)
            d = jnp.where(s_v[v] == 4, 0.0, d_v[v])
            o_v[v] = jnp.where(oh_v[v] > ot_v[v], d, -d)
            return carry

        lax.fori_loop(0, SUBL // 16, body, 0)
        wb.append(pltpu.async_copy(
            o_v.at[pl.ds(c * SUBL, SUBL)],
            signed.at[pl.ds(base + c * SUBL, SUBL)], sem_out))
    for cp in wb:
        cp.wait()

    plsc.subcore_barrier()

    f_v[...] = jnp.ones((16,), jnp.int32)
    pltpu.sync_copy(f_v, flags.at[pl.ds(sc * 16, 16)])

    other = pl.multiple_of((1 - sc) * 16, 16)
    f_v[...] = jnp.zeros((16,), jnp.int32)

    def _poll(i, carry):
        @pl.when(f_v[pl.ds(0, 16)][0] == 0)
        def _again():
            pltpu.async_copy(flags.at[pl.ds(other, 16)], f_v, sem_in).wait()
        return carry

    lax.fori_loop(0, 128, _poll, 0)

    gpend = []
    for l in range(4):
        icp[l].wait()
        idx = idx_v.at[pl.ds(l * NODE_CHUNK, NODE_CHUNK)]
        dst = g_v.at[pl.ds(l * NODE_CHUNK, NODE_CHUNK)]
        gpend.append(pltpu.async_copy(signed.at[idx], dst, gsem[l]))

    for l in range(4):
        gpend[l].wait()
        if l == 3:
            cp_m.wait()
        gb = l * NODE_CHUNK

        def bodyb(j, carry, l=l, gb=gb):
            v = pl.ds(j * 16, 16)
            g = g_v[pl.ds(gb + j * 16, 16)]
            if l == 0:
                r_v[v] = g
            elif l == 3:
                r_v[v] = r_v[v] + g - m_v[v]
            else:
                r_v[v] = r_v[v] + g
            return carry

        lax.fori_loop(0, NODE_CHUNK // 16, bodyb, 0)

    pltpu.sync_copy(r_v, out.at[pl.ds(nbase, NODE_CHUNK)])


@functools.cache
def _fused_call():
    return pl.kernel(
        _body,
        out_type=(jax.ShapeDtypeStruct((N,), jnp.float32),
                  jax.ShapeDtypeStruct((E,), jnp.float32),
                  jax.ShapeDtypeStruct((NC * 16,), jnp.int32)),
        mesh=_mesh(),
        scratch_types=[
            pltpu.VMEM((LINK_CHUNK,), jnp.float32),
            pltpu.VMEM((LINK_CHUNK,), jnp.int32),
            pltpu.VMEM((LINK_CHUNK,), jnp.int32),
            pltpu.VMEM((LINK_CHUNK,), jnp.int32),
            pltpu.VMEM((LINK_CHUNK,), jnp.float32),
            pltpu.VMEM((LINK_CHUNK,), jnp.float32),
            pltpu.VMEM((LINK_CHUNK,), jnp.float32),
            pltpu.VMEM((NODE_CHUNK * 4,), jnp.int32),
            pltpu.VMEM((NODE_CHUNK * 4,), jnp.float32),
            pltpu.VMEM((NODE_CHUNK,), jnp.float32),
            pltpu.VMEM((NODE_CHUNK,), jnp.float32),
            pltpu.VMEM((16,), jnp.int32),
            pltpu.SemaphoreType.DMA,
            pltpu.SemaphoreType.DMA,
            pltpu.SemaphoreType.DMA,
            pltpu.SemaphoreType.DMA,
            pltpu.SemaphoreType.DMA,
            pltpu.SemaphoreType.DMA,
            pltpu.SemaphoreType.DMA,
        ],
    )


def kernel(discharge, overburden, melt_rate, status_at_link,
           node_at_link_head, node_at_link_tail, links_at_node):
    status = status_at_link.astype(jnp.int32)
    head = node_at_link_head.astype(jnp.int32)
    tail = node_at_link_tail.astype(jnp.int32)
    links_t = links_at_node.astype(jnp.int32).T.reshape(4 * N)
    out, _signed, _flags = _fused_call()(
        discharge, status, head, tail, overburden, links_t, melt_rate)
    return out


# speedup vs baseline: 1.9578x; 1.0103x over previous
import functools

import jax
import jax.numpy as jnp
from jax import lax
from jax.experimental import pallas as pl
from jax.experimental.pallas import tpu as pltpu
from jax.experimental.pallas import tpu_sc as plsc

N = 100000
E = 200000
NC = 2
NS = 16
NW = NC * NS

LINK_CHUNK = 6272
NSUB = 4
SUBL = LINK_CHUNK // NSUB
HALF_N = N // 2
NODE_CHUNK = 3136


@functools.cache
def _mesh():
    return plsc.VectorSubcoreMesh(core_axis_name="c", subcore_axis_name="s",
                                  num_cores=NC, num_subcores=NS)


def _body(disch, status, head, tail, over, links_t, melt,
          out, signed, flags,
          d_v, s_v, h_v, t_v, oh_v, ot_v, o_v,
          idx_v, g_v, m_v, r_v, f_v,
          sem_in, sem_b, g0, g1, g2, g3, sem_out):
    sc = lax.axis_index("c")
    sid = lax.axis_index("s")
    wid = sid * NC + sc

    f_v[...] = jnp.zeros((16,), jnp.int32)
    pltpu.sync_copy(f_v, flags.at[pl.ds(sc * 16, 16)])

    base = jnp.minimum(wid * LINK_CHUNK, E - LINK_CHUNK)
    base = pl.multiple_of(base, 8)
    sl = pl.ds(base, LINK_CHUNK)
    cp_h = pltpu.async_copy(head.at[sl], h_v, sem_in)
    cp_t = pltpu.async_copy(tail.at[sl], t_v, sem_in)
    cp_d = pltpu.async_copy(disch.at[sl], d_v, sem_in)
    cp_s = pltpu.async_copy(status.at[sl], s_v, sem_in)

    nbase = sc * HALF_N + jnp.minimum(sid * NODE_CHUNK, HALF_N - NODE_CHUNK)
    nbase = pl.multiple_of(nbase, 8)
    icp = []
    for l in range(4):
        icp.append(pltpu.async_copy(
            links_t.at[pl.ds(l * N + nbase, NODE_CHUNK)],
            idx_v.at[pl.ds(l * NODE_CHUNK, NODE_CHUNK)], sem_b))
    cp_m = pltpu.async_copy(melt.at[pl.ds(nbase, NODE_CHUNK)], m_v, sem_b)

    cp_h.wait()
    cp_t.wait()

    gsem = [g0, g1, g2, g3]
    pend = []
    for c in range(NSUB):
        cs = pl.ds(c * SUBL, SUBL)
        pend.append((
            pltpu.async_copy(over.at[h_v.at[cs]], oh_v.at[cs], gsem[c]),
            pltpu.async_copy(over.at[t_v.at[cs]], ot_v.at[cs], gsem[c]),
        ))
    cp_d.wait()
    cp_s.wait()

    wb = []
    for c in range(NSUB):
        cg1, cg2 = pend[c]
        cg1.wait()
        cg2.wait()

        def body(i, carry, c=c):
            v = pl.ds(c * SUBL + i * 16, 16)
            d = jnp.where(s_v[v] == 4, 0.0, d_v[v])
            o_v[v] = jnp.where(oh_v[v] > ot_v[v], d, -d)
            return carry

        lax.fori_loop(0, SUBL // 16, body, 0)
        wb.append(pltpu.async_copy(
            o_v.at[pl.ds(c * SUBL, SUBL)],
            signed.at[pl.ds(base + c * SUBL, SUBL)], sem_out))
    for cp in wb:
        cp.wait()

    plsc.subcore_barrier()

    f_v[...] = jnp.ones((16,), jnp.int32)
    pltpu.sync_copy(f_v, flags.at[pl.ds(sc * 16, 16)])

    other = pl.multiple_of((1 - sc) * 16, 16)
    f_v[...] = jnp.zeros((16,), jnp.int32)

    def _outer(i, carry):
        @pl.when(f_v[pl.ds(0, 16)][0] == 0)
        def _spin():
            def _inner(j, c2):
                @pl.when(f_v[pl.ds(0, 16)][0] == 0)
                def _again():
                    pltpu.async_copy(flags.at[pl.ds(other, 16)], f_v,
                                     sem_in).wait()
                return c2
            lax.fori_loop(0, 16, _inner, 0)
        return carry

    lax.fori_loop(0, 64, _outer, 0)

    gpend = []
    for l in range(4):
        icp[l].wait()
        idx = idx_v.at[pl.ds(l * NODE_CHUNK, NODE_CHUNK)]
        dst = g_v.at[pl.ds(l * NODE_CHUNK, NODE_CHUNK)]
        gpend.append(pltpu.async_copy(signed.at[idx], dst, gsem[l]))

    for l in range(4):
        gpend[l].wait()
        if l == 3:
            cp_m.wait()
        gb = l * NODE_CHUNK

        def bodyb(j, carry, l=l, gb=gb):
            v = pl.ds(j * 16, 16)
            g = g_v[pl.ds(gb + j * 16, 16)]
            if l == 0:
                r_v[v] = g
            elif l == 3:
                r_v[v] = r_v[v] + g - m_v[v]
            else:
                r_v[v] = r_v[v] + g
            return carry

        lax.fori_loop(0, NODE_CHUNK // 16, bodyb, 0)

    pltpu.sync_copy(r_v, out.at[pl.ds(nbase, NODE_CHUNK)])


@functools.cache
def _fused_call():
    return pl.kernel(
        _body,
        out_type=(jax.ShapeDtypeStruct((N,), jnp.float32),
                  jax.ShapeDtypeStruct((E,), jnp.float32),
                  jax.ShapeDtypeStruct((NC * 16,), jnp.int32)),
        mesh=_mesh(),
        scratch_types=[
            pltpu.VMEM((LINK_CHUNK,), jnp.float32),
            pltpu.VMEM((LINK_CHUNK,), jnp.int32),
            pltpu.VMEM((LINK_CHUNK,), jnp.int32),
            pltpu.VMEM((LINK_CHUNK,), jnp.int32),
            pltpu.VMEM((LINK_CHUNK,), jnp.float32),
            pltpu.VMEM((LINK_CHUNK,), jnp.float32),
            pltpu.VMEM((LINK_CHUNK,), jnp.float32),
            pltpu.VMEM((NODE_CHUNK * 4,), jnp.int32),
            pltpu.VMEM((NODE_CHUNK * 4,), jnp.float32),
            pltpu.VMEM((NODE_CHUNK,), jnp.float32),
            pltpu.VMEM((NODE_CHUNK,), jnp.float32),
            pltpu.VMEM((16,), jnp.int32),
            pltpu.SemaphoreType.DMA,
            pltpu.SemaphoreType.DMA,
            pltpu.SemaphoreType.DMA,
            pltpu.SemaphoreType.DMA,
            pltpu.SemaphoreType.DMA,
            pltpu.SemaphoreType.DMA,
            pltpu.SemaphoreType.DMA,
        ],
    )


def kernel(discharge, overburden, melt_rate, status_at_link,
           node_at_link_head, node_at_link_tail, links_at_node):
    status = status_at_link.astype(jnp.int32)
    head = node_at_link_head.astype(jnp.int32)
    tail = node_at_link_tail.astype(jnp.int32)
    links_t = links_at_node.astype(jnp.int32).T.reshape(4 * N)
    out, _signed, _flags = _fused_call()(
        discharge, status, head, tail, overburden, links_t, melt_rate)
    return out
